# initial kernel scaffold (unmeasured)
import jax
import jax.numpy as jnp
from jax import lax
from jax.experimental import pallas as pl
from jax.experimental.pallas import tpu as pltpu

N_DEV = 4


def kernel(x, w_mat, scale_x, scale_w):
    m, k_shard = x.shape
    _, n = w_mat.shape
    m_chunk = m // N_DEV

    def body(x_ref, w_ref, sx_ref, sw_ref, out_ref,
             recv_buf, send_sems, recv_sems, ag_send_sems, ag_recv_sems):
        my = lax.axis_index("i")
        left = (my + N_DEV - 1) % N_DEV
        right = (my + 1) % N_DEV

        barrier_sem = pltpu.get_barrier_semaphore()
        for nbr in [left, right]:
            pl.semaphore_signal(
                barrier_sem, inc=1,
                device_id=(nbr,), device_id_type=pl.DeviceIdType.MESH,
            )
        pl.semaphore_wait(barrier_sem, 2)

        out_ref[...] = jnp.dot(
            x_ref[...].astype(jnp.bfloat16),
            w_ref[...].astype(jnp.bfloat16),
            preferred_element_type=jnp.float32,
        )

        for s in range(N_DEV - 1):
            send_idx = (my + N_DEV - s) % N_DEV
            recv_idx = (my + 2 * N_DEV - s - 1) % N_DEV
            rdma = pltpu.make_async_remote_copy(
                src_ref=out_ref.at[pl.ds(send_idx * m_chunk, m_chunk), :],
                dst_ref=recv_buf.at[s],
                send_sem=send_sems.at[s],
                recv_sem=recv_sems.at[s],
                device_id=(right,),
                device_id_type=pl.DeviceIdType.MESH,
            )
            rdma.start()
            rdma.wait()
            acc = out_ref[pl.ds(recv_idx * m_chunk, m_chunk), :]
            out_ref[pl.ds(recv_idx * m_chunk, m_chunk), :] = acc + recv_buf[s]

        own = (my + 1) % N_DEV
        scale = sx_ref[0] * sw_ref[0]
        y = out_ref[pl.ds(own * m_chunk, m_chunk), :] * scale
        out_ref[pl.ds(own * m_chunk, m_chunk), :] = y * (
            1.0 / (1.0 + jnp.exp(-y))
        )

        for s in range(N_DEV - 1):
            send_idx = (my + 1 + N_DEV - s) % N_DEV
            rdma = pltpu.make_async_remote_copy(
                src_ref=out_ref.at[pl.ds(send_idx * m_chunk, m_chunk), :],
                dst_ref=out_ref.at[pl.ds(send_idx * m_chunk, m_chunk), :],
                send_sem=ag_send_sems.at[s],
                recv_sem=ag_recv_sems.at[s],
                device_id=(right,),
                device_id_type=pl.DeviceIdType.MESH,
            )
            rdma.start()
            rdma.wait()

    out_shape = jax.ShapeDtypeStruct((m, n), jnp.float32)
    return pl.pallas_call(
        body,
        out_shape=out_shape,
        in_specs=[
            pl.BlockSpec(memory_space=pltpu.VMEM),
            pl.BlockSpec(memory_space=pltpu.VMEM),
            pl.BlockSpec(memory_space=pltpu.SMEM),
            pl.BlockSpec(memory_space=pltpu.SMEM),
        ],
        out_specs=pl.BlockSpec(memory_space=pltpu.VMEM),
        scratch_shapes=[
            pltpu.VMEM((N_DEV - 1, m_chunk, n), jnp.float32),
            pltpu.SemaphoreType.DMA((N_DEV - 1,)),
            pltpu.SemaphoreType.DMA((N_DEV - 1,)),
            pltpu.SemaphoreType.DMA((N_DEV - 1,)),
            pltpu.SemaphoreType.DMA((N_DEV - 1,)),
        ],
        compiler_params=pltpu.CompilerParams(collective_id=0),
    )(x, w_mat, scale_x, scale_w)


# baseline (device time: 351011 ns/iter reference)
import jax
import jax.numpy as jnp
from jax import lax
from jax.experimental import pallas as pl
from jax.experimental.pallas import tpu as pltpu

N_DEV = 4


def kernel(x, w_mat, scale_x, scale_w):
    m, k_shard = x.shape
    _, n = w_mat.shape
    m_chunk = m // N_DEV

    x8 = x.astype(jnp.float8_e4m3fn)
    w8 = w_mat.astype(jnp.float8_e4m3fn)

    def body(x_ref, w_ref, sx_ref, sw_ref, out_ref,
             res_ref, rs_recv, acc_ref, rs_send_sems, rs_recv_sems,
             ag_send_sems, ag_recv_sems, credit_sem, dma_sem):
        my = lax.axis_index("i")
        left = (my + N_DEV - 1) % N_DEV
        right = (my + 1) % N_DEV

        barrier_sem = pltpu.get_barrier_semaphore()
        for nbr in [left, right]:
            pl.semaphore_signal(
                barrier_sem, inc=1,
                device_id=(nbr,), device_id_type=pl.DeviceIdType.MESH,
            )
        pl.semaphore_wait(barrier_sem, 2)

        def partial_chunk(idx):
            return jnp.dot(
                x_ref[pl.ds(idx * m_chunk, m_chunk), :], w_ref[...],
                preferred_element_type=jnp.float32,
            )

        def res_at(idx):
            return res_ref.at[pl.ds(idx * m_chunk, m_chunk), :]

        for s in range(N_DEV - 1):
            send_idx = (my + N_DEV - s) % N_DEV
            slot = s % 2
            acc = partial_chunk(send_idx)
            if s > 0:
                acc = acc + rs_recv[(s - 1) % 2].astype(jnp.float32)
                if s == 1:
                    pl.semaphore_signal(
                        credit_sem, inc=1,
                        device_id=(left,),
                        device_id_type=pl.DeviceIdType.MESH,
                    )
            res_at(send_idx)[...] = acc.astype(jnp.bfloat16)
            if s == 2:
                pl.semaphore_wait(credit_sem, 1)
            rdma = pltpu.make_async_remote_copy(
                src_ref=res_at(send_idx),
                dst_ref=rs_recv.at[slot],
                send_sem=rs_send_sems.at[slot],
                recv_sem=rs_recv_sems.at[slot],
                device_id=(right,),
                device_id_type=pl.DeviceIdType.MESH,
            )
            rdma.start()
            rdma.wait()

        own = (my + 1) % N_DEV
        scale = sx_ref[0] * sw_ref[0]
        acc = partial_chunk(own) + rs_recv[(N_DEV - 2) % 2].astype(
            jnp.float32
        )
        y = acc * scale
        z = y * (1.0 / (1.0 + jnp.exp(-y)))
        res_at(own)[...] = z.astype(jnp.bfloat16)
        acc_ref[...] = z
        copy = pltpu.make_async_copy(
            acc_ref, out_ref.at[pl.ds(own * m_chunk, m_chunk), :], dma_sem
        )
        copy.start()
        copy.wait()

        for s in range(N_DEV - 1):
            send_idx = (my + 1 + N_DEV - s) % N_DEV
            arrive_idx = (my + N_DEV - s) % N_DEV
            rdma = pltpu.make_async_remote_copy(
                src_ref=res_at(send_idx),
                dst_ref=res_at(send_idx),
                send_sem=ag_send_sems.at[s],
                recv_sem=ag_recv_sems.at[s],
                device_id=(right,),
                device_id_type=pl.DeviceIdType.MESH,
            )
            rdma.start()
            rdma.wait()
            acc_ref[...] = res_ref[
                pl.ds(arrive_idx * m_chunk, m_chunk), :
            ].astype(jnp.float32)
            copy = pltpu.make_async_copy(
                acc_ref,
                out_ref.at[pl.ds(arrive_idx * m_chunk, m_chunk), :],
                dma_sem,
            )
            copy.start()
            copy.wait()

    out_shape = jax.ShapeDtypeStruct((m, n), jnp.float32)
    return pl.pallas_call(
        body,
        out_shape=out_shape,
        in_specs=[
            pl.BlockSpec(memory_space=pltpu.VMEM),
            pl.BlockSpec(memory_space=pltpu.VMEM),
            pl.BlockSpec(memory_space=pltpu.SMEM),
            pl.BlockSpec(memory_space=pltpu.SMEM),
        ],
        out_specs=pl.BlockSpec(memory_space=pltpu.MemorySpace.HBM),
        scratch_shapes=[
            pltpu.VMEM((m, n), jnp.bfloat16),
            pltpu.VMEM((2, m_chunk, n), jnp.bfloat16),
            pltpu.VMEM((m_chunk, n), jnp.float32),
            pltpu.SemaphoreType.DMA((2,)),
            pltpu.SemaphoreType.DMA((2,)),
            pltpu.SemaphoreType.DMA((N_DEV - 1,)),
            pltpu.SemaphoreType.DMA((N_DEV - 1,)),
            pltpu.SemaphoreType.REGULAR,
            pltpu.SemaphoreType.DMA,
        ],
        compiler_params=pltpu.CompilerParams(
            collective_id=0,
            vmem_limit_bytes=56 * 1024 * 1024,
        ),
    )(x8, w8, scale_x, scale_w)


# device time: 211149 ns/iter; 1.6624x vs baseline; 1.6624x over previous
import jax
import jax.numpy as jnp
from jax import lax
from jax.experimental import pallas as pl
from jax.experimental.pallas import tpu as pltpu

N_DEV = 4


def kernel(x, w_mat, scale_x, scale_w):
    m, k_shard = x.shape
    _, n = w_mat.shape
    m_chunk = m // N_DEV
    n2 = n // 2

    x8 = x.astype(jnp.float8_e4m3fn)
    w8 = w_mat.astype(jnp.float8_e4m3fn)

    def body(x_ref, w_ref, sx_ref, sw_ref, out_ref,
             res_ref, recv_r, recv_l, acc_r, acc_l,
             rs_send_r, rs_send_l, rs_recv_r, rs_recv_l,
             ag_send_r, ag_send_l, ag_recv_r, ag_recv_l,
             credit_r, credit_l, dma_sem):
        my = lax.axis_index("i")
        left = (my + N_DEV - 1) % N_DEV
        right = (my + 1) % N_DEV

        barrier_sem = pltpu.get_barrier_semaphore()
        for nbr in [left, right]:
            pl.semaphore_signal(
                barrier_sem, inc=1,
                device_id=(nbr,), device_id_type=pl.DeviceIdType.MESH,
            )
        pl.semaphore_wait(barrier_sem, 2)

        def dot_r(idx):
            acc_r[...] = jnp.dot(
                x_ref[pl.ds(idx * m_chunk, m_chunk), :],
                w_ref[:, :n2], preferred_element_type=jnp.float32,
            )

        def dot_l(idx):
            acc_l[...] = jnp.dot(
                x_ref[pl.ds(idx * m_chunk, m_chunk), :],
                w_ref[:, n2:], preferred_element_type=jnp.float32,
            )

        def res_r(idx):
            return res_ref.at[pl.ds(idx * m_chunk, m_chunk), :n2]

        def res_l(idx):
            return res_ref.at[pl.ds(idx * m_chunk, m_chunk), n2:]

        def c(k):
            return (my + 8 * N_DEV + k) % N_DEV

        pending = []

        def send(src, dst_buf, send_sem, recv_sem, to):
            rdma = pltpu.make_async_remote_copy(
                src_ref=src, dst_ref=dst_buf,
                send_sem=send_sem, recv_sem=recv_sem,
                device_id=(to,), device_id_type=pl.DeviceIdType.MESH,
            )
            rdma.start()
            pending.append(rdma)
            return rdma

        rs = {}
        dot_r(c(0))
        dot_l(c(0))
        res_r(c(0))[...] = acc_r[...].astype(jnp.bfloat16)
        res_l(c(0))[...] = acc_l[...].astype(jnp.bfloat16)
        for s in range(N_DEV - 1):
            slot = s % 2
            rs[("r", s)] = send(res_r(c(-s)), recv_r.at[slot],
                                rs_send_r.at[s], rs_recv_r.at[slot], right)
            rs[("l", s)] = send(res_l(c(s)), recv_l.at[slot],
                                rs_send_l.at[s], rs_recv_l.at[slot], left)
            nxt = s + 1
            if nxt <= N_DEV - 1:
                dot_r(c(-nxt))
                dot_l(c(nxt))
            rs[("r", s)].wait_recv()
            rs[("l", s)].wait_recv()
            if nxt <= N_DEV - 2:
                res_r(c(-nxt))[...] = (
                    acc_r[...] + recv_r[slot].astype(jnp.float32)
                ).astype(jnp.bfloat16)
                res_l(c(nxt))[...] = (
                    acc_l[...] + recv_l[slot].astype(jnp.float32)
                ).astype(jnp.bfloat16)
            if s == 0:
                pl.semaphore_signal(
                    credit_r, inc=1, device_id=(left,),
                    device_id_type=pl.DeviceIdType.MESH,
                )
                pl.semaphore_signal(
                    credit_l, inc=1, device_id=(right,),
                    device_id_type=pl.DeviceIdType.MESH,
                )
            if s == 1:
                pl.semaphore_wait(credit_r, 1)
                pl.semaphore_wait(credit_l, 1)

        scale = sx_ref[0] * sw_ref[0]
        last = (N_DEV - 2) % 2

        y = (acc_r[...] + recv_r[last].astype(jnp.float32)) * scale
        z = y * (1.0 / (1.0 + jnp.exp(-y)))
        res_r(c(1))[...] = z.astype(jnp.bfloat16)
        acc_r[...] = z
        own_r_copy = pltpu.make_async_copy(
            acc_r, out_ref.at[pl.ds(c(1) * m_chunk, m_chunk), :n2], dma_sem
        )
        own_r_copy.start()

        y = (acc_l[...] + recv_l[last].astype(jnp.float32)) * scale
        z = y * (1.0 / (1.0 + jnp.exp(-y)))
        res_l(c(-1))[...] = z.astype(jnp.bfloat16)
        own_r_copy.wait()
        acc_l[...] = z
        own_l_copy = pltpu.make_async_copy(
            acc_l, out_ref.at[pl.ds(c(-1) * m_chunk, m_chunk), n2:], dma_sem
        )
        own_l_copy.start()
        own_l_copy.wait()

        for s in range(N_DEV - 1):
            ar = send(res_r(c(1 - s)), res_r(c(1 - s)),
                      ag_send_r.at[s], ag_recv_r.at[s], right)
            al = send(res_l(c(s - 1)), res_l(c(s - 1)),
                      ag_send_l.at[s], ag_recv_l.at[s], left)
            ar.wait_recv()
            acc_r[...] = res_ref[
                pl.ds(c(-s) * m_chunk, m_chunk), :n2
            ].astype(jnp.float32)
            cp = pltpu.make_async_copy(
                acc_r, out_ref.at[pl.ds(c(-s) * m_chunk, m_chunk), :n2],
                dma_sem,
            )
            cp.start()
            al.wait_recv()
            acc_l[...] = res_ref[
                pl.ds(c(s) * m_chunk, m_chunk), n2:
            ].astype(jnp.float32)
            cp.wait()
            cp = pltpu.make_async_copy(
                acc_l, out_ref.at[pl.ds(c(s) * m_chunk, m_chunk), n2:],
                dma_sem,
            )
            cp.start()
            cp.wait()

        for rdma in pending:
            rdma.wait_send()

    out_shape = jax.ShapeDtypeStruct((m, n), jnp.float32)
    return pl.pallas_call(
        body,
        out_shape=out_shape,
        in_specs=[
            pl.BlockSpec(memory_space=pltpu.VMEM),
            pl.BlockSpec(memory_space=pltpu.VMEM),
            pl.BlockSpec(memory_space=pltpu.SMEM),
            pl.BlockSpec(memory_space=pltpu.SMEM),
        ],
        out_specs=pl.BlockSpec(memory_space=pltpu.MemorySpace.HBM),
        scratch_shapes=[
            pltpu.VMEM((m, n), jnp.bfloat16),
            pltpu.VMEM((2, m_chunk, n2), jnp.bfloat16),
            pltpu.VMEM((2, m_chunk, n2), jnp.bfloat16),
            pltpu.VMEM((m_chunk, n2), jnp.float32),
            pltpu.VMEM((m_chunk, n2), jnp.float32),
            pltpu.SemaphoreType.DMA((N_DEV - 1,)),
            pltpu.SemaphoreType.DMA((N_DEV - 1,)),
            pltpu.SemaphoreType.DMA((2,)),
            pltpu.SemaphoreType.DMA((2,)),
            pltpu.SemaphoreType.DMA((N_DEV - 1,)),
            pltpu.SemaphoreType.DMA((N_DEV - 1,)),
            pltpu.SemaphoreType.DMA((N_DEV - 1,)),
            pltpu.SemaphoreType.DMA((N_DEV - 1,)),
            pltpu.SemaphoreType.REGULAR,
            pltpu.SemaphoreType.REGULAR,
            pltpu.SemaphoreType.DMA,
        ],
        compiler_params=pltpu.CompilerParams(
            collective_id=0,
            vmem_limit_bytes=56 * 1024 * 1024,
        ),
    )(x8, w8, scale_x, scale_w)


# device time: 183212 ns/iter; 1.9159x vs baseline; 1.1525x over previous
import jax
import jax.numpy as jnp
from jax import lax
from jax.experimental import pallas as pl
from jax.experimental.pallas import tpu as pltpu

N_DEV = 4


def kernel(x, w_mat, scale_x, scale_w):
    m, k_shard = x.shape
    _, n = w_mat.shape
    kh = k_shard // 2
    mc = m // N_DEV

    x8 = x.astype(jnp.float8_e4m3fn)
    w8 = w_mat.astype(jnp.float8_e4m3fn)

    def body(x_ref, w_ref, sx_ref, sw_ref, out_ref,
             xf, wf, acc, tmp,
             xs_r, ws_r, xs_l, ws_l,
             xr_r, wr_r, xr_l, wr_l,
             dma_sem):
        my = lax.axis_index("i")
        left = (my + N_DEV - 1) % N_DEV
        right = (my + 1) % N_DEV

        def c(k):
            return (my + 8 * N_DEV + k) % N_DEV

        barrier_sem = pltpu.get_barrier_semaphore()
        for nbr in [left, right]:
            pl.semaphore_signal(
                barrier_sem, inc=1,
                device_id=(nbr,), device_id_type=pl.DeviceIdType.MESH,
            )
        pl.semaphore_wait(barrier_sem, 2)

        pending = []

        def send(src, dst, ssem, rsem, to):
            rdma = pltpu.make_async_remote_copy(
                src_ref=src, dst_ref=dst, send_sem=ssem, recv_sem=rsem,
                device_id=(to,), device_id_type=pl.DeviceIdType.MESH,
            )
            rdma.start()
            pending.append(rdma)
            return rdma

        def accum(j, h, init):
            for r in range(N_DEV):
                d = jnp.dot(
                    xf[j, h, pl.ds(r * mc, mc), :], wf[j, h],
                    preferred_element_type=jnp.float32,
                )
                if init:
                    acc[pl.ds(r * mc, mc), :] = d.astype(jnp.bfloat16)
                else:
                    acc[pl.ds(r * mc, mc), :] = (
                        acc[pl.ds(r * mc, mc), :].astype(jnp.float32) + d
                    ).astype(jnp.bfloat16)

        recvs = []
        for s in range(N_DEV - 1):
            if s == 0:
                sr_x, sr_w = x_ref.at[:, :kh], w_ref.at[pl.ds(0, kh), :]
                sl_x, sl_w = x_ref.at[:, kh:], w_ref.at[pl.ds(kh, kh), :]
            else:
                sr_x, sr_w = xf.at[c(-s), 0], wf.at[c(-s), 0]
                sl_x, sl_w = xf.at[c(s), 1], wf.at[c(s), 1]
            step = [
                send(sr_x, xf.at[c(-s), 0], xs_r.at[s], xr_r.at[s], right),
                send(sr_w, wf.at[c(-s), 0], ws_r.at[s], wr_r.at[s], right),
                send(sl_x, xf.at[c(s), 1], xs_l.at[s], xr_l.at[s], left),
                send(sl_w, wf.at[c(s), 1], ws_l.at[s], wr_l.at[s], left),
            ]
            if s == 0:
                for r in range(N_DEV):
                    acc[pl.ds(r * mc, mc), :] = jnp.dot(
                        x_ref[pl.ds(r * mc, mc), :kh], w_ref[pl.ds(0, kh), :],
                        preferred_element_type=jnp.float32,
                    ).astype(jnp.bfloat16)
                for r in range(N_DEV):
                    acc[pl.ds(r * mc, mc), :] = (
                        acc[pl.ds(r * mc, mc), :].astype(jnp.float32)
                        + jnp.dot(
                            x_ref[pl.ds(r * mc, mc), kh:],
                            w_ref[pl.ds(kh, kh), :],
                            preferred_element_type=jnp.float32,
                        )
                    ).astype(jnp.bfloat16)
            else:
                accum(c(-s), 0, False)
                accum(c(s), 1, False)
            for rdma in step:
                rdma.wait_recv()
            recvs.append(step)

        scale = sx_ref[0] * sw_ref[0]
        prev_cp = None
        for r in range(N_DEV):
            t = (
                acc[pl.ds(r * mc, mc), :].astype(jnp.float32)
                + jnp.dot(
                    xf[c(1), 0, pl.ds(r * mc, mc), :], wf[c(1), 0],
                    preferred_element_type=jnp.float32,
                )
                + jnp.dot(
                    xf[c(-1), 1, pl.ds(r * mc, mc), :], wf[c(-1), 1],
                    preferred_element_type=jnp.float32,
                )
            )
            y = t * scale
            z = y * (1.0 / (1.0 + jnp.exp(-y)))
            if prev_cp is not None:
                prev_cp.wait()
            tmp[...] = z
            prev_cp = pltpu.make_async_copy(
                tmp, out_ref.at[pl.ds(r * mc, mc), :], dma_sem
            )
            prev_cp.start()
        prev_cp.wait()

        for rdma in pending:
            rdma.wait_send()

    out_shape = jax.ShapeDtypeStruct((m, n), jnp.float32)
    return pl.pallas_call(
        body,
        out_shape=out_shape,
        in_specs=[
            pl.BlockSpec(memory_space=pltpu.VMEM),
            pl.BlockSpec(memory_space=pltpu.VMEM),
            pl.BlockSpec(memory_space=pltpu.SMEM),
            pl.BlockSpec(memory_space=pltpu.SMEM),
        ],
        out_specs=pl.BlockSpec(memory_space=pltpu.MemorySpace.HBM),
        scratch_shapes=[
            pltpu.VMEM((N_DEV, 2, m, kh), jnp.float8_e4m3fn),
            pltpu.VMEM((N_DEV, 2, kh, n), jnp.float8_e4m3fn),
            pltpu.VMEM((m, n), jnp.bfloat16),
            pltpu.VMEM((mc, n), jnp.float32),
            pltpu.SemaphoreType.DMA((N_DEV - 1,)),
            pltpu.SemaphoreType.DMA((N_DEV - 1,)),
            pltpu.SemaphoreType.DMA((N_DEV - 1,)),
            pltpu.SemaphoreType.DMA((N_DEV - 1,)),
            pltpu.SemaphoreType.DMA((N_DEV - 1,)),
            pltpu.SemaphoreType.DMA((N_DEV - 1,)),
            pltpu.SemaphoreType.DMA((N_DEV - 1,)),
            pltpu.SemaphoreType.DMA((N_DEV - 1,)),
            pltpu.SemaphoreType.DMA,
        ],
        compiler_params=pltpu.CompilerParams(
            collective_id=0,
            vmem_limit_bytes=62 * 1024 * 1024,
        ),
    )(x8, w8, scale_x, scale_w)
